# Initial kernel scaffold; baseline (speedup 1.0000x reference)
import jax
import jax.numpy as jnp
from jax.experimental import pallas as pl


def kernel(x, w1, g1, b1, m1, v1, w2, g2, b2, m2, v2):
    raise NotImplementedError("write your pallas kernel here")


# speedup vs baseline: 1.4345x; 1.4345x over previous
import jax
import jax.numpy as jnp
from jax.experimental import pallas as pl
from jax.experimental.pallas import tpu as pltpu


def _rup(x, m):
    return (x + m - 1) // m * m


def _fold_bn(gamma, beta, mean, var, eps=1e-5):
    scale = gamma / jnp.sqrt(var + eps)
    return scale, beta - mean * scale


def _make_body(C, NB, P_pad, MARGIN, L, base_shifts):
    G = len(base_shifts)
    tail = L - MARGIN - P_pad

    def _build_b9(src, b9_ref):
        for gi, s in enumerate(base_shifts):
            shift = (-(s + MARGIN - 128)) % L
            b9_ref[gi * C:(gi + 1) * C, :] = \
                pltpu.roll(src, shift=shift, axis=1)[:, :NB]

    def _combine(y_ref):
        return (y_ref[0:C, 127:127 + P_pad] +
                y_ref[C:2 * C, 128:128 + P_pad] +
                y_ref[2 * C:3 * C, 129:129 + P_pad])

    def body(x_ref, wa_ref, ba_ref, wb_ref, bb_ref, m_ref, o_ref,
             b9_ref, y_ref, h_ref):
        x = x_ref[0]

        _build_b9(x, b9_ref)
        y_ref[...] = jnp.dot(wa_ref[...], b9_ref[...],
                             preferred_element_type=jnp.float32)
        h = jnp.maximum(_combine(y_ref) + ba_ref[...], 0.0) * m_ref[...]
        h_ref[:, 0:MARGIN] = jnp.zeros((C, MARGIN), jnp.bfloat16)
        h_ref[:, MARGIN + P_pad:L] = jnp.zeros((C, tail), jnp.bfloat16)
        h_ref[:, MARGIN:MARGIN + P_pad] = h.astype(jnp.bfloat16)

        _build_b9(h_ref[...], b9_ref)
        y_ref[...] = jnp.dot(wb_ref[...], b9_ref[...],
                             preferred_element_type=jnp.float32)
        res = x[:, MARGIN:MARGIN + P_pad].astype(jnp.float32)
        o_ref[0] = jnp.maximum(_combine(y_ref) + bb_ref[...] + res, 0.0)

    return body


def kernel(x, w1, g1, b1, m1, v1, w2, g2, b2, m2, v2):
    N, C, D, H, W = x.shape
    Dp, Hp, Wp = D + 2, H + 2, W + 2
    P = Dp * Hp * Wp
    M0 = Hp * Wp + Wp + 1
    MARGIN = _rup(M0, 128)
    P_pad = _rup(P, 128)
    L = MARGIN + P_pad + MARGIN
    NB = P_pad + 256
    assert NB <= L and MARGIN >= M0

    scale1, bias1 = _fold_bn(g1, b1, m1, v1)
    scale2, bias2 = _fold_bn(g2, b2, m2, v2)
    w1s = w1 * scale1[:, None, None, None, None]
    w2s = w2 * scale2[:, None, None, None, None]
    wa = jnp.transpose(w1s, (4, 0, 2, 3, 1)).reshape(3 * C, 9 * C)
    wb = jnp.transpose(w2s, (4, 0, 2, 3, 1)).reshape(3 * C, 9 * C)
    wa = wa.astype(jnp.bfloat16)
    wb = wb.astype(jnp.bfloat16)
    ba = bias1.reshape(C, 1).astype(jnp.float32)
    bb = bias2.reshape(C, 1).astype(jnp.float32)

    xp = jnp.pad(x, ((0, 0), (0, 0), (1, 1), (1, 1), (1, 1)))
    x_ext = jnp.pad(xp.reshape(N, C, P),
                    ((0, 0), (0, 0), (MARGIN, L - MARGIN - P)))
    x_ext = x_ext.astype(jnp.bfloat16)

    interior = jnp.zeros((Dp, Hp, Wp), jnp.float32)
    interior = interior.at[1:1 + D, 1:1 + H, 1:1 + W].set(1.0)
    mask = jnp.pad(interior.reshape(1, P), ((0, 0), (0, P_pad - P)))

    base_shifts = [(kd - 1) * Hp * Wp + (kh - 1) * Wp
                   for kd in range(3) for kh in range(3)]

    body = _make_body(C, NB, P_pad, MARGIN, L, base_shifts)

    flops = 2 * (2 * 27 * C * C * P_pad) * N
    bytes_accessed = int(2 * x_ext.size + 4 * (N * C * P_pad + mask.size)
                         + 2 * (wa.size + wb.size))

    out_flat = pl.pallas_call(
        body,
        out_shape=jax.ShapeDtypeStruct((N, C, P_pad), jnp.float32),
        grid=(N,),
        in_specs=[
            pl.BlockSpec((1, C, L), lambda n: (n, 0, 0)),
            pl.BlockSpec((3 * C, 9 * C), lambda n: (0, 0)),
            pl.BlockSpec((C, 1), lambda n: (0, 0)),
            pl.BlockSpec((3 * C, 9 * C), lambda n: (0, 0)),
            pl.BlockSpec((C, 1), lambda n: (0, 0)),
            pl.BlockSpec((1, P_pad), lambda n: (0, 0)),
        ],
        out_specs=pl.BlockSpec((1, C, P_pad), lambda n: (n, 0, 0)),
        scratch_shapes=[
            pltpu.VMEM((9 * C, NB), jnp.bfloat16),
            pltpu.VMEM((3 * C, NB), jnp.float32),
            pltpu.VMEM((C, L), jnp.bfloat16),
        ],
        compiler_params=pltpu.CompilerParams(
            dimension_semantics=("parallel",),
            vmem_limit_bytes=64 * 1024 * 1024),
        cost_estimate=pl.CostEstimate(
            flops=flops, transcendentals=0, bytes_accessed=bytes_accessed),
    )(x_ext, wa, ba, wb, bb, mask)

    out = out_flat[:, :, :P].reshape(N, C, Dp, Hp, Wp)
    return out[:, :, 1:1 + D, 1:1 + H, 1:1 + W]


# speedup vs baseline: 3.5014x; 2.4408x over previous
import jax
import jax.numpy as jnp
from jax.experimental import pallas as pl
from jax.experimental.pallas import tpu as pltpu


def _rup(x, m):
    return (x + m - 1) // m * m


def _fold_bn(gamma, beta, mean, var, eps=1e-5):
    scale = gamma / jnp.sqrt(var + eps)
    return scale, beta - mean * scale


OFF = 128


def _make_body(C, P0, NB, MARGIN, L, base_shifts):
    def _build_b9(src, vm_ref, b9_ref):
        for gi, s in enumerate(base_shifts):
            shift = (-(s + MARGIN - OFF)) % L
            b9_ref[gi * C:(gi + 1) * C, :] = \
                pltpu.roll(src, shift=shift, axis=1)[:, :NB] * vm_ref[gi:gi + 1, :]

    def _combine(y_ref, wm_ref):
        return (y_ref[0:C, OFF - 1:OFF - 1 + P0] * wm_ref[0:1, :] +
                y_ref[C:2 * C, OFF:OFF + P0] +
                y_ref[2 * C:3 * C, OFF + 1:OFF + 1 + P0] * wm_ref[1:2, :])

    def body(x_ref, wa_ref, ba_ref, wb_ref, bb_ref, vm_ref, wm_ref, o_ref,
             b9_ref, y_ref, f_ref):
        f_ref[:, 0:MARGIN] = jnp.zeros((C, MARGIN), jnp.bfloat16)
        f_ref[:, MARGIN + P0:L] = jnp.zeros((C, L - MARGIN - P0), jnp.bfloat16)
        f_ref[:, MARGIN:MARGIN + P0] = x_ref[0].astype(jnp.bfloat16)

        _build_b9(f_ref[...], vm_ref, b9_ref)
        y_ref[...] = jnp.dot(wa_ref[...], b9_ref[...],
                             preferred_element_type=jnp.float32)
        h = jnp.maximum(_combine(y_ref, wm_ref) + ba_ref[...], 0.0)
        f_ref[:, MARGIN:MARGIN + P0] = h.astype(jnp.bfloat16)

        _build_b9(f_ref[...], vm_ref, b9_ref)
        y_ref[...] = jnp.dot(wb_ref[...], b9_ref[...],
                             preferred_element_type=jnp.float32)
        o_ref[0] = jnp.maximum(
            _combine(y_ref, wm_ref) + bb_ref[...] + x_ref[0], 0.0)

    return body


def kernel(x, w1, g1, b1, m1, v1, w2, g2, b2, m2, v2):
    N, C, D, H, W = x.shape
    P0 = D * H * W
    M0 = H * W + W + 1
    MARGIN = _rup(M0, 128)
    L = MARGIN + P0 + MARGIN
    NB = P0 + 2 * OFF
    assert NB <= L and MARGIN >= M0

    scale1, bias1 = _fold_bn(g1, b1, m1, v1)
    scale2, bias2 = _fold_bn(g2, b2, m2, v2)
    w1s = w1 * scale1[:, None, None, None, None]
    w2s = w2 * scale2[:, None, None, None, None]
    wa = jnp.transpose(w1s, (4, 0, 2, 3, 1)).reshape(3 * C, 9 * C)
    wb = jnp.transpose(w2s, (4, 0, 2, 3, 1)).reshape(3 * C, 9 * C)
    wa = wa.astype(jnp.bfloat16)
    wb = wb.astype(jnp.bfloat16)
    ba = bias1.reshape(C, 1).astype(jnp.float32)
    bb = bias2.reshape(C, 1).astype(jnp.float32)

    j0 = jnp.arange(NB) - OFF
    in_range = (j0 >= 0) & (j0 < P0)
    d0 = jnp.clip(j0, 0, P0 - 1) // (H * W)
    h0 = (jnp.clip(j0, 0, P0 - 1) // W) % H
    vrows = []
    for kd in range(3):
        for kh in range(3):
            ok = (in_range & (d0 + kd - 1 >= 0) & (d0 + kd - 1 < D)
                  & (h0 + kh - 1 >= 0) & (h0 + kh - 1 < H))
            vrows.append(ok)
    vmask = jnp.stack(vrows + [jnp.zeros((NB,), bool)] * 7)
    vmask = vmask.astype(jnp.bfloat16)
    wj = jnp.arange(P0) % W
    wmask = jnp.stack([(wj >= 1)] + [(wj <= W - 2)] + [jnp.zeros((P0,), bool)] * 6)
    wmask = wmask.astype(jnp.float32)

    base_shifts = [(kd - 1) * H * W + (kh - 1) * W
                   for kd in range(3) for kh in range(3)]

    body = _make_body(C, P0, NB, MARGIN, L, base_shifts)

    x_flat = x.reshape(N, C, P0)

    flops = 2 * (2 * 27 * C * C * P0) * N
    bytes_accessed = int(4 * x_flat.size + 4 * (N * C * P0)
                         + 2 * (wa.size + wb.size))

    out_flat = pl.pallas_call(
        body,
        out_shape=jax.ShapeDtypeStruct((N, C, P0), jnp.float32),
        grid=(N,),
        in_specs=[
            pl.BlockSpec((1, C, P0), lambda n: (n, 0, 0)),
            pl.BlockSpec((3 * C, 9 * C), lambda n: (0, 0)),
            pl.BlockSpec((C, 1), lambda n: (0, 0)),
            pl.BlockSpec((3 * C, 9 * C), lambda n: (0, 0)),
            pl.BlockSpec((C, 1), lambda n: (0, 0)),
            pl.BlockSpec((16, NB), lambda n: (0, 0)),
            pl.BlockSpec((8, P0), lambda n: (0, 0)),
        ],
        out_specs=pl.BlockSpec((1, C, P0), lambda n: (n, 0, 0)),
        scratch_shapes=[
            pltpu.VMEM((9 * C, NB), jnp.bfloat16),
            pltpu.VMEM((3 * C, NB), jnp.float32),
            pltpu.VMEM((C, L), jnp.bfloat16),
        ],
        compiler_params=pltpu.CompilerParams(
            dimension_semantics=("parallel",),
            vmem_limit_bytes=64 * 1024 * 1024),
        cost_estimate=pl.CostEstimate(
            flops=flops, transcendentals=0, bytes_accessed=bytes_accessed),
    )(x_flat, wa, ba, wb, bb, vmask, wmask)

    return out_flat.reshape(N, C, D, H, W)
